# Initial kernel scaffold; baseline (speedup 1.0000x reference)
import functools

import jax
import jax.numpy as jnp
from jax import lax
from jax.experimental import pallas as pl
from jax.experimental.pallas import tpu as pltpu
from jax.experimental.pallas import tpu_sc as plsc

NUM_ROWS = 16384 * 50
DIM = 64
NC, NS = 2, 16
NW = NC * NS
ROWS_PER_W = NUM_ROWS // NW
IDX_MINOR = 128
NB = 4
CHUNK = NB * IDX_MINOR
N_CHUNKS = ROWS_PER_W // CHUNK


def _body(idx_hbm, table_hbm, out_hbm, idx_v, rows_v, sem_g):
    wid = lax.axis_index("s") * NC + lax.axis_index("c")
    base = wid * ROWS_PER_W

    @pl.loop(0, N_CHUNKS)
    def _chunk(c):
        off = base + c * CHUNK
        pltpu.sync_copy(idx_hbm.at[pl.ds(off, CHUNK)], idx_v)
        for j in range(NB):
            pltpu.async_copy(
                table_hbm.at[idx_v.at[j]],
                rows_v.at[pl.ds(j * IDX_MINOR, IDX_MINOR)],
                sem_g,
            )
        for j in range(NB):
            pltpu.make_async_copy(
                table_hbm.at[idx_v.at[j]],
                rows_v.at[pl.ds(j * IDX_MINOR, IDX_MINOR)],
                sem_g,
            ).wait()
        pltpu.sync_copy(rows_v, out_hbm.at[pl.ds(off, CHUNK)])


@jax.jit
def _gather(table, idx):
    mesh = plsc.VectorSubcoreMesh(core_axis_name="c", subcore_axis_name="s")
    f = functools.partial(
        pl.kernel,
        out_type=jax.ShapeDtypeStruct((NUM_ROWS, DIM), jnp.float32),
        mesh=mesh,
        scratch_types=[
            pltpu.VMEM((NB, IDX_MINOR), jnp.int32),
            pltpu.VMEM((CHUNK, DIM), jnp.float32),
            pltpu.SemaphoreType.DMA,
        ],
    )(_body)
    return f(idx.reshape(NW * N_CHUNKS * NB, IDX_MINOR), table)


def kernel(entity_indices, table):
    idx = entity_indices.reshape(-1).astype(jnp.int32)
    out = _gather(table, idx)
    return out.reshape(entity_indices.shape + (DIM,))


# speedup vs baseline: 1.7990x; 1.7990x over previous
import functools

import jax
import jax.numpy as jnp
from jax import lax
from jax.experimental import pallas as pl
from jax.experimental.pallas import tpu as pltpu
from jax.experimental.pallas import tpu_sc as plsc

NUM_ROWS = 16384 * 50
DIM = 64
NC, NS = 2, 16
NW = NC * NS
ROWS_PER_W = NUM_ROWS // NW
IDX_MINOR = 128
NB = 4
CHUNK = NB * IDX_MINOR
N_CHUNKS = ROWS_PER_W // CHUNK


def _body(idx_hbm, table_hbm, out_hbm, idx_v, rows_v, sem_g):
    wid = lax.axis_index("s") * NC + lax.axis_index("c")
    base = wid * ROWS_PER_W
    base_g = wid * (ROWS_PER_W // IDX_MINOR)

    @pl.loop(0, N_CHUNKS)
    def _chunk(c):
        off = base + c * CHUNK
        pltpu.sync_copy(idx_hbm.at[pl.ds(base_g + c * NB, NB)], idx_v)
        for j in range(NB):
            pltpu.async_copy(
                table_hbm.at[idx_v.at[j]],
                rows_v.at[pl.ds(j * IDX_MINOR, IDX_MINOR)],
                sem_g,
            )
        for j in range(NB):
            pltpu.make_async_copy(
                table_hbm.at[idx_v.at[j]],
                rows_v.at[pl.ds(j * IDX_MINOR, IDX_MINOR)],
                sem_g,
            ).wait()
        pltpu.sync_copy(rows_v, out_hbm.at[pl.ds(off, CHUNK)])


@jax.jit
def _gather(table, idx):
    mesh = plsc.VectorSubcoreMesh(core_axis_name="c", subcore_axis_name="s")
    f = functools.partial(
        pl.kernel,
        out_type=jax.ShapeDtypeStruct((NUM_ROWS, DIM), jnp.float32),
        mesh=mesh,
        scratch_types=[
            pltpu.VMEM((NB, IDX_MINOR), jnp.int32),
            pltpu.VMEM((CHUNK, DIM), jnp.float32),
            pltpu.SemaphoreType.DMA,
        ],
        compiler_params=pltpu.CompilerParams(use_tc_tiling_on_sc=False),
    )(_body)
    return f(idx.reshape(NW * N_CHUNKS * NB, IDX_MINOR), table)


def kernel(entity_indices, table):
    idx = entity_indices.reshape(-1).astype(jnp.int32)
    out = _gather(table, idx)
    return out.reshape(entity_indices.shape + (DIM,))


# speedup vs baseline: 1.8735x; 1.0414x over previous
import functools

import jax
import jax.numpy as jnp
from jax import lax
from jax.experimental import pallas as pl
from jax.experimental.pallas import tpu as pltpu
from jax.experimental.pallas import tpu_sc as plsc

NUM_ROWS = 16384 * 50
DIM = 64
NC, NS = 2, 16
NW = NC * NS
ROWS_PER_W = NUM_ROWS // NW
IDX_MINOR = 128
NB = 4
CHUNK = NB * IDX_MINOR
N_CHUNKS = ROWS_PER_W // CHUNK
GROUPS_PER_W = ROWS_PER_W // IDX_MINOR


def _body(idx_hbm, table_hbm, out_hbm, idx_v, rows_v, sem_g, sem_w0, sem_w1):
    wid = lax.axis_index("s") * NC + lax.axis_index("c")
    base = wid * ROWS_PER_W
    base_g = wid * GROUPS_PER_W
    sems_w = (sem_w0, sem_w1)

    pltpu.sync_copy(idx_hbm.at[pl.ds(base_g, GROUPS_PER_W)], idx_v)

    def fire_gathers(c, b):
        for j in range(NB):
            pltpu.async_copy(
                table_hbm.at[idx_v.at[c * NB + j]],
                rows_v.at[b, pl.ds(j * IDX_MINOR, IDX_MINOR)],
                sem_g,
            )

    def drain_gathers(c, b):
        for j in range(NB):
            pltpu.make_async_copy(
                table_hbm.at[idx_v.at[c * NB + j]],
                rows_v.at[b, pl.ds(j * IDX_MINOR, IDX_MINOR)],
                sem_g,
            ).wait()

    def writeback(c, b):
        return pltpu.make_async_copy(
            rows_v.at[b], out_hbm.at[pl.ds(base + c * CHUNK, CHUNK)], sems_w[b]
        )

    for b in range(2):
        fire_gathers(b, b)
        drain_gathers(b, b)
        writeback(b, b).start()

    @pl.loop(1, N_CHUNKS // 2)
    def _pair(p):
        for b in range(2):
            c = 2 * p + b
            writeback(c - 2, b).wait()
            fire_gathers(c, b)
            drain_gathers(c, b)
            writeback(c, b).start()

    for b in range(2):
        writeback(N_CHUNKS - 2 + b, b).wait()


@jax.jit
def _gather(table, idx):
    mesh = plsc.VectorSubcoreMesh(core_axis_name="c", subcore_axis_name="s")
    f = functools.partial(
        pl.kernel,
        out_type=jax.ShapeDtypeStruct((NUM_ROWS, DIM), jnp.float32),
        mesh=mesh,
        scratch_types=[
            pltpu.VMEM((GROUPS_PER_W, IDX_MINOR), jnp.int32),
            pltpu.VMEM((2, CHUNK, DIM), jnp.float32),
            pltpu.SemaphoreType.DMA,
            pltpu.SemaphoreType.DMA,
            pltpu.SemaphoreType.DMA,
        ],
        compiler_params=pltpu.CompilerParams(use_tc_tiling_on_sc=False),
    )(_body)
    return f(idx.reshape(NUM_ROWS // IDX_MINOR, IDX_MINOR), table)


def kernel(entity_indices, table):
    idx = entity_indices.reshape(-1).astype(jnp.int32)
    out = _gather(table, idx)
    return out.reshape(entity_indices.shape + (DIM,))
